# baseline (speedup 1.0000x reference)
import functools

import jax
import jax.numpy as jnp
from jax import lax
from jax.experimental import pallas as pl
from jax.experimental.pallas import tpu as pltpu
from jax.experimental.pallas import tpu_sc as plsc

_TOP_K = 8
_N_EXP = 64


def _gate_body(x_ref, wt_ref, probs_ref):
    logits = jnp.dot(x_ref[...], wt_ref[...], preferred_element_type=jnp.float32)
    m = jnp.max(logits, axis=-1, keepdims=True)
    e = jnp.exp(logits - m)
    s = jnp.sum(e, axis=-1, keepdims=True)
    probs_ref[...] = e / s


def _gate_probs(xf, wt, tile, sz=None, off=0):
    n, h = xf.shape
    n_exp = wt.shape[1]
    if sz is None:
        sz = n
    blk0 = off // tile
    return pl.pallas_call(
        _gate_body,
        grid=(sz // tile,),
        in_specs=[
            pl.BlockSpec((tile, h), lambda i: (blk0 + i, 0)),
            pl.BlockSpec((h, n_exp), lambda i: (0, 0)),
        ],
        out_specs=pl.BlockSpec((tile, n_exp), lambda i: (i, 0)),
        out_shape=jax.ShapeDtypeStruct((sz, n_exp), jnp.float32),
    )(xf, wt)


def _topk_sc_body(probs_hbm, w_hbm, idx_hbm, slab, wv, iv, *, rows_per, nc):
    wid = lax.axis_index("s") * nc + lax.axis_index("c")
    row0 = wid * rows_per
    pltpu.sync_copy(probs_hbm.at[pl.ds(row0 * _N_EXP, rows_per * _N_EXP)], slab)
    lane = lax.iota(jnp.int32, 16)

    def group(g, carry):
        for gg in range(2):
            rows = (2 * g + gg) * 16 + lane
            vbase = rows * _N_EXP
            obase = rows * _TOP_K
            tv = [jnp.full((16,), -1.0, jnp.float32) for _ in range(_TOP_K)]
            ti = [jnp.zeros((16,), jnp.int32) for _ in range(_TOP_K)]
            for e in range(_N_EXP):
                cur_v = plsc.load_gather(slab, [vbase + e])
                cur_i = jnp.full((16,), e, jnp.int32)
                for j in range(_TOP_K):
                    gt = cur_v > tv[j]
                    new_v = jnp.where(gt, cur_v, tv[j])
                    new_i = jnp.where(gt, cur_i, ti[j])
                    cur_v = jnp.where(gt, tv[j], cur_v)
                    cur_i = jnp.where(gt, ti[j], cur_i)
                    tv[j] = new_v
                    ti[j] = new_i
            ssum = tv[0]
            for j in range(1, _TOP_K):
                ssum = ssum + tv[j]
            for j in range(_TOP_K):
                plsc.store_scatter(wv, [obase + j], tv[j] / ssum)
                plsc.store_scatter(iv, [obase + j], ti[j])
        return carry

    lax.fori_loop(0, rows_per // 32, group, 0)
    pltpu.sync_copy(wv, w_hbm.at[pl.ds(row0 * _TOP_K, rows_per * _TOP_K)])
    pltpu.sync_copy(iv, idx_hbm.at[pl.ds(row0 * _TOP_K, rows_per * _TOP_K)])


def _topk_sc(probs):
    n = probs.shape[0]
    info = plsc.get_sparse_core_info()
    nw = info.num_cores * info.num_subcores
    rows_per = n // nw
    fn = functools.partial(_topk_sc_body, rows_per=rows_per, nc=info.num_cores)
    wf, idxf = pl.kernel(
        fn,
        mesh=plsc.VectorSubcoreMesh(core_axis_name="c", subcore_axis_name="s"),
        compiler_params=pltpu.CompilerParams(needs_layout_passes=False),
        out_type=[
            jax.ShapeDtypeStruct((n * _TOP_K,), jnp.float32),
            jax.ShapeDtypeStruct((n * _TOP_K,), jnp.int32),
        ],
        scratch_types=[
            pltpu.VMEM((rows_per * _N_EXP,), jnp.float32),
            pltpu.VMEM((rows_per * _TOP_K,), jnp.float32),
            pltpu.VMEM((rows_per * _TOP_K,), jnp.int32),
        ],
    )(probs.reshape(-1))
    return wf.reshape(n, _TOP_K), idxf.reshape(n, _TOP_K)


def kernel(x, W):
    b, s, h = x.shape
    n = b * s
    xf = x.reshape(n, h)
    wt = W.T

    nchunk = 4
    while n % (nchunk * 1024):
        nchunk //= 2
    sz = n // nchunk

    probs_c, w_c, idx_c = [], [], []
    for c in range(nchunk):
        probs_c.append(_gate_probs(xf, wt, tile=1024, sz=sz, off=c * sz))
    for c in range(nchunk):
        wc, ic = _topk_sc(probs_c[c])
        w_c.append(wc)
        idx_c.append(ic)

    probs = jnp.concatenate(probs_c, axis=0)
    w = jnp.concatenate(w_c, axis=0)
    idx = jnp.concatenate(idx_c, axis=0)

    return (
        w.reshape(b, s, _TOP_K),
        idx.reshape(b, s, _TOP_K),
        probs.reshape(b, s, _N_EXP),
    )


# speedup vs baseline: 1.0738x; 1.0738x over previous
import functools

import jax
import jax.numpy as jnp
from jax import lax
from jax.experimental import pallas as pl
from jax.experimental.pallas import tpu as pltpu
from jax.experimental.pallas import tpu_sc as plsc

_TOP_K = 8
_N_EXP = 64


def _softmax(logits):
    m = jnp.max(logits, axis=-1, keepdims=True)
    e = jnp.exp(logits - m)
    s = jnp.sum(e, axis=-1, keepdims=True)
    return e / s


def _top8(probs):
    lane = jax.lax.broadcasted_iota(jnp.int32, probs.shape, dimension=1)
    work = probs
    ws, idxs = [], []
    for _ in range(_TOP_K):
        mx = jnp.max(work, axis=-1, keepdims=True)
        is_max = work == mx
        cand = jnp.where(is_max, lane, _N_EXP)
        sel = jnp.min(cand, axis=-1, keepdims=True)
        ws.append(mx)
        idxs.append(sel)
        work = jnp.where(lane == sel, -1.0, work)
    w = jnp.concatenate(ws, axis=1)
    idx = jnp.concatenate(idxs, axis=1)
    w = w / jnp.sum(w, axis=-1, keepdims=True)
    return w, idx


def _gate_body(x_ref, wt_ref, probs_ref):
    logits = jnp.dot(x_ref[...], wt_ref[...], preferred_element_type=jnp.float32)
    probs_ref[...] = _softmax(logits)


def _fused_body(x_ref, wt_ref, probs_ref, w_ref, i_ref):
    logits = jnp.dot(x_ref[...], wt_ref[...], preferred_element_type=jnp.float32)
    probs = _softmax(logits)
    probs_ref[...] = probs
    w, idx = _top8(probs)
    w_ref[...] = w
    i_ref[...] = idx


def _gate_probs(xf, wt, tile, sz, off):
    n, h = xf.shape
    n_exp = wt.shape[1]
    blk0 = off // tile
    return pl.pallas_call(
        _gate_body,
        grid=(sz // tile,),
        in_specs=[
            pl.BlockSpec((tile, h), lambda i: (blk0 + i, 0)),
            pl.BlockSpec((h, n_exp), lambda i: (0, 0)),
        ],
        out_specs=pl.BlockSpec((tile, n_exp), lambda i: (i, 0)),
        out_shape=jax.ShapeDtypeStruct((sz, n_exp), jnp.float32),
    )(xf, wt)


def _gate_fused(xf, wt, tile, sz, off):
    n, h = xf.shape
    n_exp = wt.shape[1]
    blk0 = off // tile
    return pl.pallas_call(
        _fused_body,
        grid=(sz // tile,),
        in_specs=[
            pl.BlockSpec((tile, h), lambda i: (blk0 + i, 0)),
            pl.BlockSpec((h, n_exp), lambda i: (0, 0)),
        ],
        out_specs=[
            pl.BlockSpec((tile, n_exp), lambda i: (i, 0)),
            pl.BlockSpec((tile, _TOP_K), lambda i: (i, 0)),
            pl.BlockSpec((tile, _TOP_K), lambda i: (i, 0)),
        ],
        out_shape=[
            jax.ShapeDtypeStruct((sz, n_exp), jnp.float32),
            jax.ShapeDtypeStruct((sz, _TOP_K), jnp.float32),
            jax.ShapeDtypeStruct((sz, _TOP_K), jnp.int32),
        ],
    )(xf, wt)


def _topk_sc_body(probs_hbm, w_hbm, idx_hbm, slab, wv, iv, *, rows_per, nc):
    wid = lax.axis_index("s") * nc + lax.axis_index("c")
    row0 = wid * rows_per
    pltpu.sync_copy(probs_hbm.at[pl.ds(row0 * _N_EXP, rows_per * _N_EXP)], slab)
    lane = lax.iota(jnp.int32, 16)

    def group(g, carry):
        for gg in range(2):
            rows = (2 * g + gg) * 16 + lane
            vbase = rows * _N_EXP
            obase = rows * _TOP_K
            tv = [jnp.full((16,), -1.0, jnp.float32) for _ in range(_TOP_K)]
            ti = [jnp.zeros((16,), jnp.int32) for _ in range(_TOP_K)]
            for e in range(_N_EXP):
                cur_v = plsc.load_gather(slab, [vbase + e])
                cur_i = jnp.full((16,), e, jnp.int32)
                for j in range(_TOP_K):
                    gt = cur_v > tv[j]
                    new_v = jnp.where(gt, cur_v, tv[j])
                    new_i = jnp.where(gt, cur_i, ti[j])
                    cur_v = jnp.where(gt, tv[j], cur_v)
                    cur_i = jnp.where(gt, ti[j], cur_i)
                    tv[j] = new_v
                    ti[j] = new_i
            ssum = tv[0]
            for j in range(1, _TOP_K):
                ssum = ssum + tv[j]
            for j in range(_TOP_K):
                plsc.store_scatter(wv, [obase + j], tv[j] / ssum)
                plsc.store_scatter(iv, [obase + j], ti[j])
        return carry

    lax.fori_loop(0, rows_per // 32, group, 0)
    pltpu.sync_copy(wv, w_hbm.at[pl.ds(row0 * _TOP_K, rows_per * _TOP_K)])
    pltpu.sync_copy(iv, idx_hbm.at[pl.ds(row0 * _TOP_K, rows_per * _TOP_K)])


def _topk_sc(probs):
    n = probs.shape[0]
    info = plsc.get_sparse_core_info()
    nw = info.num_cores * info.num_subcores
    rows_per = n // nw
    fn = functools.partial(_topk_sc_body, rows_per=rows_per, nc=info.num_cores)
    wf, idxf = pl.kernel(
        fn,
        mesh=plsc.VectorSubcoreMesh(core_axis_name="c", subcore_axis_name="s"),
        compiler_params=pltpu.CompilerParams(needs_layout_passes=False),
        out_type=[
            jax.ShapeDtypeStruct((n * _TOP_K,), jnp.float32),
            jax.ShapeDtypeStruct((n * _TOP_K,), jnp.int32),
        ],
        scratch_types=[
            pltpu.VMEM((rows_per * _N_EXP,), jnp.float32),
            pltpu.VMEM((rows_per * _TOP_K,), jnp.float32),
            pltpu.VMEM((rows_per * _TOP_K,), jnp.int32),
        ],
    )(probs.reshape(-1))
    return wf.reshape(n, _TOP_K), idxf.reshape(n, _TOP_K)


def kernel(x, W):
    b, s, h = x.shape
    n = b * s
    xf = x.reshape(n, h)
    wt = W.T

    tile = 1024
    while n % tile:
        tile //= 2

    sc_rows = 2048
    if n % sc_rows or n <= sc_rows or sc_rows % 32 or tile > sc_rows:
        sc_rows = 0

    if sc_rows:
        p0 = _gate_probs(xf, wt, tile, sz=sc_rows, off=0)
        w0, i0 = _topk_sc(p0)
        p1, w1, i1 = _gate_fused(xf, wt, tile, sz=n - sc_rows, off=sc_rows)
        probs = jnp.concatenate([p0, p1], axis=0)
        w = jnp.concatenate([w0, w1], axis=0)
        idx = jnp.concatenate([i0, i1], axis=0)
    else:
        probs, w, idx = _gate_fused(xf, wt, tile, sz=n, off=0)

    return (
        w.reshape(b, s, _TOP_K),
        idx.reshape(b, s, _TOP_K),
        probs.reshape(b, s, _N_EXP),
    )
